# baseline (speedup 1.0000x reference)
import functools

import jax
import jax.numpy as jnp
from jax.experimental import pallas as pl
from jax.experimental.pallas import tpu as pltpu

K = 32
QB = 64
CH = 512


def _knn_body(xyz2t_ref, xyz1_ref, out_ref, d_ref):
    n2 = xyz2t_ref.shape[2]
    nch = n2 // CH
    x1 = xyz1_ref[0]
    qx = x1[:, 0:1]
    qy = x1[:, 1:2]
    qz = x1[:, 2:3]
    yy = qx * qx + qy * qy + qz * qz
    r = lambda v: v.astype(jnp.bfloat16).astype(jnp.float32)
    qxb, qyb, qzb = r(qx), r(qy), r(qz)

    for c in range(nch):
        cx = xyz2t_ref[0, 0:1, c * CH:(c + 1) * CH]
        cy = xyz2t_ref[0, 1:2, c * CH:(c + 1) * CH]
        cz = xyz2t_ref[0, 2:3, c * CH:(c + 1) * CH]
        xx = cx * cx + cy * cy + cz * cz
        dot = qxb * r(cx) + qyb * r(cy) + qzb * r(cz)
        d_ref[c] = jnp.maximum((xx + yy) - 2.0 * dot, 0.0)

    iota_c = jax.lax.broadcasted_iota(jnp.int32, (QB, CH), 1)
    col_iota = jax.lax.broadcasted_iota(jnp.int32, (QB, K), 1)
    big_i = jnp.int32(n2)
    inf = jnp.float32(jnp.inf)

    def find(j, carry):
        pv, pi, acc = carry

        def chunk(c, ch_carry):
            bv, bi = ch_carry
            d = d_ref[c]
            idx = iota_c + c * CH
            live = (d > pv) | ((d == pv) & (idx > pi))
            dm = jnp.where(live, d, inf)
            cv = jnp.min(dm, axis=1, keepdims=True)
            ci = jnp.min(jnp.where(dm == cv, idx, big_i), axis=1,
                         keepdims=True)
            take = (cv < bv) | ((cv == bv) & (ci < bi))
            return jnp.where(take, cv, bv), jnp.where(take, ci, bi)

        bv, bi = jax.lax.fori_loop(0, nch, chunk,
                                   (jnp.full((QB, 1), inf),
                                    jnp.full((QB, 1), big_i)))
        acc = jnp.where(col_iota == (j - 1), bi, acc)
        return bv, bi, acc

    init = (jnp.full((QB, 1), -jnp.float32(1.0)),
            jnp.full((QB, 1), jnp.int32(-1)),
            jnp.zeros((QB, K), jnp.int32))
    _, _, acc = jax.lax.fori_loop(0, K + 1, find, init)
    out_ref[0] = acc


@jax.jit
def kernel(xyz2, xyz1):
    b, n2, _ = xyz2.shape
    n1 = xyz1.shape[1]
    xyz2t = jnp.transpose(xyz2, (0, 2, 1))
    grid = (b, n1 // QB)
    return pl.pallas_call(
        _knn_body,
        grid=grid,
        in_specs=[
            pl.BlockSpec((1, 3, n2), lambda i, q: (i, 0, 0)),
            pl.BlockSpec((1, QB, 3), lambda i, q: (i, q, 0)),
        ],
        out_specs=pl.BlockSpec((1, QB, K), lambda i, q: (i, q, 0)),
        out_shape=jax.ShapeDtypeStruct((b, n1, K), jnp.int32),
        scratch_shapes=[pltpu.VMEM((n2 // CH, QB, CH), jnp.float32)],
    )(xyz2t, xyz1)


# speedup vs baseline: 6.0795x; 6.0795x over previous
import functools

import jax
import jax.numpy as jnp
from jax import lax
from jax.experimental import pallas as pl
from jax.experimental.pallas import tpu as pltpu
from jax.experimental.pallas import tpu_sc as plsc

K = 32
N2 = 4096
NQW = 512
NV = N2 // 16
NB = 144
CAP = 48
QBATCH = 8

_GDN = lax.GatherDimensionNumbers(
    offset_dims=(), collapsed_slice_dims=(0,), start_index_map=(0,))


def _take16(v, idx):
    return lax.gather(v, idx[:, None], _GDN, (1,),
                      mode=lax.GatherScatterMode.PROMISE_IN_BOUNDS)


def _rne_bf16(v):
    i32 = jnp.int32
    bits = lax.bitcast_convert_type(v, i32)
    r = bits + 0x7FFF + (lax.shift_right_logical(bits, 16) & 1)
    return lax.bitcast_convert_type(r & i32(-65536), jnp.float32)


def _sc_body(c_h, q_h, out_h,
             cf_v, cr_v, xx_v, qf_v, qrs_v, qyy_v,
             d_v, hist_v, tot_v, sd_v, si_v, tmp_v, ob_v):
    i32 = jnp.int32
    cid = lax.axis_index("c")
    sid = lax.axis_index("s")
    wid = sid * 2 + cid
    b = wid // 8
    qoff = (wid % 8) * NQW
    cbase = b * 3 * N2

    lane = lax.iota(i32, 16)
    ones = jnp.ones((16,), i32)

    pltpu.sync_copy(c_h.at[pl.ds(cbase, 3 * N2)], cf_v)
    for c in range(3):
        pltpu.sync_copy(q_h.at[pl.ds(cbase + c * N2 + qoff, NQW)],
                        qf_v.at[pl.ds(c * NQW, NQW)])

    def xx_loop(i, _):
        x = cf_v[pl.ds(i * 16, 16)]
        y = cf_v[pl.ds(N2 + i * 16, 16)]
        z = cf_v[pl.ds(2 * N2 + i * 16, 16)]
        xx_v[pl.ds(i * 16, 16)] = (x * x + y * y) + z * z
        cr_v[pl.ds(i * 16, 16)] = -2.0 * _rne_bf16(x)
        cr_v[pl.ds(N2 + i * 16, 16)] = -2.0 * _rne_bf16(y)
        cr_v[pl.ds(2 * N2 + i * 16, 16)] = -2.0 * _rne_bf16(z)
        return 0
    lax.fori_loop(0, NV, xx_loop, 0)

    def hclear(i, _):
        hist_v[pl.ds(i * 16, 16)] = jnp.zeros((16,), i32)
        return 0
    lax.fori_loop(0, QBATCH * 16 * NB // 16, hclear, 0)

    def qyy_loop(i, _):
        x = qf_v[pl.ds(i * 16, 16)]
        y = qf_v[pl.ds(NQW + i * 16, 16)]
        z = qf_v[pl.ds(2 * NQW + i * 16, 16)]
        qyy_v[pl.ds(i * 16, 16)] = (x * x + y * y) + z * z
        qrs_v[pl.ds(i * 16, 16)] = _rne_bf16(x)
        qrs_v[pl.ds(NQW + i * 16, 16)] = _rne_bf16(y)
        qrs_v[pl.ds(2 * NQW + i * 16, 16)] = _rne_bf16(z)
        return 0
    lax.fori_loop(0, NQW // 16, qyy_loop, 0)

    def blk_loop(qb, _):
        q0 = qb * QBATCH
        qblk = (q0 // 16) * 16
        qrel = q0 - qblk
        qxv = qrs_v[pl.ds(qblk, 16)]
        qyv = qrs_v[pl.ds(NQW + qblk, 16)]
        qzv = qrs_v[pl.ds(2 * NQW + qblk, 16)]
        qyyv = qyy_v[pl.ds(qblk, 16)]
        qbs = []
        for u in range(QBATCH):
            qsel = jnp.full((16,), qrel + u, i32)
            qbs.append((_take16(qxv, qsel), _take16(qyv, qsel),
                        _take16(qzv, qsel), _take16(qyyv, qsel)))

        def dist_loop(i2, _):
            for w in range(2):
                i = i2 * 2 + w
                cx = cr_v[pl.ds(i * 16, 16)]
                cy = cr_v[pl.ds(N2 + i * 16, 16)]
                cz = cr_v[pl.ds(2 * N2 + i * 16, 16)]
                xxc = xx_v[pl.ds(i * 16, 16)]
                for u in range(QBATCH):
                    qxb, qyb, qzb, qyyb = qbs[u]
                    s = (qxb * cx + qyb * cy) + qzb * cz
                    d = jnp.maximum((xxc + qyyb) + s, 0.0)
                    d_v[pl.ds(u * N2 + i * 16, 16)] = d
                    key = lax.shift_right_logical(
                        lax.bitcast_convert_type(d, i32), 23)
                    plsc.addupdate_scatter(
                        hist_v, [(u * 16 + lane) * NB + key], ones)
            return 0
        lax.fori_loop(0, NV // 2, dist_loop, 0)

        def q_loop(u, _):
            q = q0 + u
            hb = u * 16 * NB
            db = u * N2

            zeros16 = jnp.zeros((16,), i32)
            for jc in range(NB // 16):
                acc = zeros16
                for l in range(16):
                    acc = acc + hist_v[pl.ds(hb + l * NB + jc * 16, 16)]
                    hist_v[pl.ds(hb + l * NB + jc * 16, 16)] = zeros16
                tot_v[pl.ds(jc * 16, 16)] = acc

            csums = jnp.zeros((16,), i32)
            for jc in range(NB // 16):
                s = jnp.sum(tot_v[pl.ds(jc * 16, 16)])
                csums = jnp.where(lane == jc, s, csums)
            cumc = plsc.cumsum(csums)
            hitc = (cumc >= 33).astype(i32)
            jstar = 16 - jnp.sum(plsc.cummax(hitc))
            cum_before = jnp.sum(jnp.where(lane == jstar - 1, cumc, 0))
            tot = tot_v[pl.ds(jstar * 16, 16)]
            cumv = plsc.cumsum(tot) + cum_before
            hit = (cumv >= 33).astype(i32)
            nbefore = 16 - jnp.sum(plsc.cummax(hit))
            strictC = jnp.sum(jnp.where(lane == nbefore - 1, cumv, 0))
            strictC = jnp.where(nbefore == 0, cum_before, strictC)
            eqC = jnp.sum(jnp.where(lane == nbefore, tot, 0))
            P = jstar * 16 + nbefore

            def rcond(st):
                shift, P, strictC, eqC = st
                return (strictC + eqC > CAP) & (shift > 3)

            def refine(st):
                shift, P, strictC, eqC = st
                nshift = shift - 4

                def hloop(i, _):
                    bits = lax.bitcast_convert_type(
                        d_v[pl.ds(db + i * 16, 16)], i32)
                    m = lax.shift_right_logical(bits, shift) == P
                    digit = lax.shift_right_logical(bits, nshift) & 15
                    plsc.addupdate_scatter(
                        hist_v, [hb + lane * NB + digit], ones, mask=m)
                    return 0
                lax.fori_loop(0, NV, hloop, 0)

                acc = jnp.zeros((16,), i32)
                for l in range(16):
                    acc = acc + hist_v[pl.ds(hb + l * NB, 16)]
                    hist_v[pl.ds(hb + l * NB, 16)] = jnp.zeros((16,), i32)
                cumv = plsc.cumsum(acc) + strictC
                hit = (cumv >= 33).astype(i32)
                nbefore = 16 - jnp.sum(plsc.cummax(hit))
                strict_new = jnp.sum(
                    jnp.where(lane == nbefore - 1, cumv, 0))
                strict_new = jnp.where(nbefore == 0, strictC, strict_new)
                eq_new = jnp.sum(jnp.where(lane == nbefore, acc, 0))
                return (nshift, P * 16 + nbefore, strict_new, eq_new)

            shift, P, strictC, eqC = lax.while_loop(
                rcond, refine, (i32(23), P, strictC, eqC))

            for j in range(4):
                sd_v[pl.ds(j * 16, 16)] = jnp.full((16,), jnp.inf,
                                                   jnp.float32)
                si_v[pl.ds(j * 16, 16)] = N2 + j * 16 + lane

            def comp(i2, cnt):
                for w in range(2):
                    i = i2 * 2 + w
                    d = d_v[pl.ds(db + i * 16, 16)]
                    bits = lax.bitcast_convert_type(d, i32)
                    m = lax.shift_right_logical(bits, shift) <= P
                    base = jnp.minimum(cnt, i32(CAP))
                    plsc.store_compressed(sd_v.at[pl.ds(base, 16)], d,
                                          mask=m)
                    plsc.store_compressed(si_v.at[pl.ds(base, 16)],
                                          i * 16 + lane, mask=m)
                    cnt = jnp.minimum(cnt + jnp.sum(m.astype(i32)),
                                      i32(CAP))
                return cnt
            lax.fori_loop(0, NV // 2, comp, i32(0))

            svd = [sd_v[pl.ds(j * 16, 16)] for j in range(3)]
            svi = [si_v[pl.ds(j * 16, 16)] for j in range(3)]

            def rank_loop(r, ranks):
                rsel = (lane + r) & 15
                rots = [(_take16(svd[bb], rsel), _take16(svi[bb], rsel))
                        for bb in range(3)]
                out = []
                for a in range(3):
                    da, ia, rk = svd[a], svi[a], ranks[a]
                    for bb in range(3):
                        db_, ib = rots[bb]
                        lt = (db_ < da) | ((db_ == da) & (ib < ia))
                        rk = rk + jnp.where(lt, 1, 0)
                    out.append(rk)
                return tuple(out)
            ranks = lax.fori_loop(0, 16, rank_loop,
                                  (zeros16, zeros16, zeros16))
            for a in range(3):
                rank_a = ranks[a]
                msk = (rank_a >= 1) & (rank_a <= K)
                plsc.store_scatter(tmp_v, [rank_a - 1], svi[a], mask=msk)

            ob_v[pl.ds(q * K, 16)] = tmp_v[pl.ds(0, 16)]
            ob_v[pl.ds(q * K + 16, 16)] = tmp_v[pl.ds(16, 16)]
            return 0

        lax.fori_loop(0, QBATCH, q_loop, 0)
        return 0

    lax.fori_loop(0, NQW // QBATCH, blk_loop, 0)

    pltpu.sync_copy(ob_v, out_h.at[pl.ds(wid * NQW * K, NQW * K)])


@jax.jit
def kernel(xyz2, xyz1):
    b, n2, _ = xyz2.shape
    n1 = xyz1.shape[1]
    c_soa = jnp.transpose(xyz2, (0, 2, 1)).reshape(-1)
    q_soa = jnp.transpose(xyz1, (0, 2, 1)).reshape(-1)

    mesh = plsc.VectorSubcoreMesh(core_axis_name="c", subcore_axis_name="s")
    f32 = jnp.float32
    i32 = jnp.int32
    run = pl.kernel(
        _sc_body,
        out_type=jax.ShapeDtypeStruct((b * n1 * K,), i32),
        mesh=mesh,
        compiler_params=pltpu.CompilerParams(needs_layout_passes=False),
        scratch_types=[
            pltpu.VMEM((3 * N2,), f32),
            pltpu.VMEM((3 * N2,), f32),
            pltpu.VMEM((N2,), f32),
            pltpu.VMEM((3 * NQW,), f32),
            pltpu.VMEM((3 * NQW,), f32),
            pltpu.VMEM((NQW,), f32),
            pltpu.VMEM((QBATCH * N2,), f32),
            pltpu.VMEM((QBATCH * 16 * NB,), i32),
            pltpu.VMEM((NB,), i32),
            pltpu.VMEM((CAP + 16,), f32),
            pltpu.VMEM((CAP + 16,), i32),
            pltpu.VMEM((CAP + 16,), i32),
            pltpu.VMEM((NQW * K,), i32),
        ],
    )
    out = run(c_soa, q_soa)
    return out.reshape(b, n1, K)
